# speedup vs baseline: 5.5174x; 5.5174x over previous
import functools
import math

import jax
import jax.numpy as jnp
from jax import lax
from jax.experimental import pallas as pl
from jax.experimental.pallas import tpu as pltpu
from jax.experimental.pallas import tpu_sc as plsc

_NC = 2
_NS = 16
_NW = _NC * _NS
_G = 128


def _scale_body(t_ref, o_ref, *, scale):
    o_ref[...] = t_ref[...] * scale


def _scale_table(table):
    v, d = table.shape
    blk = 1000
    assert v % blk == 0
    return pl.pallas_call(
        functools.partial(_scale_body, scale=math.sqrt(d)),
        out_shape=jax.ShapeDtypeStruct((v, d), table.dtype),
        grid=(v // blk,),
        in_specs=[pl.BlockSpec((blk, d), lambda i: (i, 0))],
        out_specs=pl.BlockSpec((blk, d), lambda i: (i, 0)),
    )(table)


@functools.cache
def _make_gather(t, d):
    assert t % (_NW * _G) == 0
    npw = t // _NW
    ng = npw // _G

    mesh = plsc.VectorSubcoreMesh(
        core_axis_name="c", subcore_axis_name="s",
        num_cores=_NC, num_subcores=_NS)

    @functools.partial(
        pl.kernel,
        out_type=jax.ShapeDtypeStruct((t, d), jnp.float32),
        mesh=mesh,
        scratch_types=[
            pltpu.VMEM((ng, _G), jnp.int32),
            pltpu.VMEM((_G, d), jnp.float32),
            pltpu.SemaphoreType.DMA,
        ],
    )
    def gather(table_hbm, idx_hbm, out_hbm, idx_v, rows_v, sem):
        wid = lax.axis_index("s") * _NC + lax.axis_index("c")
        pltpu.sync_copy(idx_hbm.at[pl.ds(wid * ng, ng), :], idx_v)

        def step(j, carry):
            pltpu.async_copy(table_hbm.at[idx_v.at[j]], rows_v, sem).wait()
            pltpu.sync_copy(rows_v, out_hbm.at[pl.ds(wid * npw + j * _G, _G), :])
            return carry

        lax.fori_loop(0, ng, step, 0)

    return gather


def kernel(tok_ids, emb_table):
    b, l = tok_ids.shape
    v, d = emb_table.shape
    t = b * l
    scaled = _scale_table(emb_table)
    idx = tok_ids.reshape(t // _G, _G).astype(jnp.int32)
    out = _make_gather(t, d)(scaled, idx)
    return out.reshape(b, l, d)


# speedup vs baseline: 7.4683x; 1.3536x over previous
import functools
import math

import jax
import jax.numpy as jnp
from jax import lax
from jax.experimental import pallas as pl
from jax.experimental.pallas import tpu as pltpu
from jax.experimental.pallas import tpu_sc as plsc

_NC = 2
_NS = 16
_NW = _NC * _NS
_G = 128


def _scale_body(t_ref, o_ref, *, scale):
    o_ref[...] = t_ref[...] * scale


def _scale_table(table):
    v, d = table.shape
    blk = 1000
    assert v % blk == 0
    return pl.pallas_call(
        functools.partial(_scale_body, scale=math.sqrt(d)),
        out_shape=jax.ShapeDtypeStruct((v, d), table.dtype),
        grid=(v // blk,),
        in_specs=[pl.BlockSpec((blk, d), lambda i: (i, 0))],
        out_specs=pl.BlockSpec((blk, d), lambda i: (i, 0)),
    )(table)


_NBUF = 4


@functools.cache
def _make_gather(t, d):
    assert t % (_NW * _G * _NBUF) == 0
    npw = t // _NW
    ng = npw // _G

    mesh = plsc.VectorSubcoreMesh(
        core_axis_name="c", subcore_axis_name="s",
        num_cores=_NC, num_subcores=_NS)

    @functools.partial(
        pl.kernel,
        out_type=jax.ShapeDtypeStruct((t, d), jnp.float32),
        mesh=mesh,
        scratch_types=[
            pltpu.VMEM((ng, _G), jnp.int32),
            [pltpu.VMEM((_G, d), jnp.float32)] * _NBUF,
            [pltpu.SemaphoreType.DMA] * _NBUF,
            [pltpu.SemaphoreType.DMA] * _NBUF,
        ],
    )
    def gather(table_hbm, idx_hbm, out_hbm, idx_v, bufs, gsems, ssems):
        wid = lax.axis_index("s") * _NC + lax.axis_index("c")
        pltpu.sync_copy(idx_hbm.at[pl.ds(wid * ng, ng), :], idx_v)

        def out_at(j):
            return out_hbm.at[pl.ds(wid * npw + j * _G, _G), :]

        for b in range(_NBUF):
            pltpu.async_copy(table_hbm.at[idx_v.at[b]], bufs[b], gsems[b])

        def outer(k, carry):
            g0 = k * _NBUF
            for b in range(_NBUF):
                j = g0 + b
                pltpu.make_async_copy(
                    table_hbm.at[idx_v.at[j]], bufs[b], gsems[b]).wait()
                pltpu.async_copy(bufs[b], out_at(j), ssems[b])
            for b in range(_NBUF):
                j2 = g0 + _NBUF + b

                @pl.when(j2 < ng)
                def _():
                    pltpu.make_async_copy(bufs[b], out_at(g0 + b), ssems[b]).wait()
                    pltpu.async_copy(table_hbm.at[idx_v.at[j2]], bufs[b], gsems[b])

            return carry

        lax.fori_loop(0, ng // _NBUF, outer, 0)

        for b in range(_NBUF):
            pltpu.make_async_copy(bufs[b], out_at(ng - _NBUF + b), ssems[b]).wait()

    return gather


def kernel(tok_ids, emb_table):
    b, l = tok_ids.shape
    v, d = emb_table.shape
    t = b * l
    scaled = _scale_table(emb_table)
    idx = tok_ids.reshape(t // _G, _G).astype(jnp.int32)
    out = _make_gather(t, d)(scaled, idx)
    return out.reshape(b, l, d)


# speedup vs baseline: 9.0872x; 1.2168x over previous
import functools
import math

import jax
import jax.numpy as jnp
from jax import lax
from jax.experimental import pallas as pl
from jax.experimental.pallas import tpu as pltpu
from jax.experimental.pallas import tpu_sc as plsc

_NC = 2
_NS = 16
_NW = _NC * _NS
_G = 128


_NBUF = 4


@functools.cache
def _make_gather(t, d):
    assert t % (_NW * _G * _NBUF) == 0
    npw = t // _NW
    ng = npw // _G

    mesh = plsc.VectorSubcoreMesh(
        core_axis_name="c", subcore_axis_name="s",
        num_cores=_NC, num_subcores=_NS)

    @functools.partial(
        pl.kernel,
        out_type=jax.ShapeDtypeStruct((t, d), jnp.float32),
        mesh=mesh,
        scratch_types=[
            pltpu.VMEM((ng, _G), jnp.int32),
            [pltpu.VMEM((_G, d), jnp.float32)] * _NBUF,
            [pltpu.SemaphoreType.DMA] * _NBUF,
            [pltpu.SemaphoreType.DMA] * _NBUF,
        ],
    )
    def gather(table_hbm, idx_hbm, out_hbm, idx_v, bufs, gsems, ssems):
        wid = lax.axis_index("s") * _NC + lax.axis_index("c")
        pltpu.sync_copy(idx_hbm.at[pl.ds(wid * ng, ng), :], idx_v)

        def out_at(j):
            return out_hbm.at[pl.ds(wid * npw + j * _G, _G), :]

        for b in range(_NBUF):
            pltpu.async_copy(table_hbm.at[idx_v.at[b]], bufs[b], gsems[b])

        scale = jnp.float32(math.sqrt(d))

        def scale_buf(buf):
            def row(i, carry):
                for cc in range(d // 16):
                    sl = (i, pl.ds(cc * 16, 16))
                    buf[sl] = buf[sl] * scale
                return carry

            lax.fori_loop(0, _G, row, 0)

        def outer(k, carry):
            g0 = k * _NBUF
            for b in range(_NBUF):
                j = g0 + b
                pltpu.make_async_copy(
                    table_hbm.at[idx_v.at[j]], bufs[b], gsems[b]).wait()
                scale_buf(bufs[b])
                pltpu.async_copy(bufs[b], out_at(j), ssems[b])
            for b in range(_NBUF):
                j2 = g0 + _NBUF + b

                @pl.when(j2 < ng)
                def _():
                    pltpu.make_async_copy(bufs[b], out_at(g0 + b), ssems[b]).wait()
                    pltpu.async_copy(table_hbm.at[idx_v.at[j2]], bufs[b], gsems[b])

            return carry

        lax.fori_loop(0, ng // _NBUF, outer, 0)

        for b in range(_NBUF):
            pltpu.make_async_copy(bufs[b], out_at(ng - _NBUF + b), ssems[b]).wait()

    return gather


def kernel(tok_ids, emb_table):
    b, l = tok_ids.shape
    v, d = emb_table.shape
    t = b * l
    idx = tok_ids.reshape(t // _G, _G).astype(jnp.int32)
    out = _make_gather(t, d)(emb_table, idx)
    return out.reshape(b, l, d)


# speedup vs baseline: 9.1030x; 1.0017x over previous
import functools
import math

import jax
import jax.numpy as jnp
from jax import lax
from jax.experimental import pallas as pl
from jax.experimental.pallas import tpu as pltpu
from jax.experimental.pallas import tpu_sc as plsc

_NC = 2
_NS = 16
_NW = _NC * _NS
_G = 128


_NBUF = 5


@functools.cache
def _make_gather(t, d):
    assert t % (_NW * _G * _NBUF) == 0
    npw = t // _NW
    ng = npw // _G

    mesh = plsc.VectorSubcoreMesh(
        core_axis_name="c", subcore_axis_name="s",
        num_cores=_NC, num_subcores=_NS)

    @functools.partial(
        pl.kernel,
        out_type=jax.ShapeDtypeStruct((t, d), jnp.float32),
        mesh=mesh,
        scratch_types=[
            pltpu.VMEM((ng, _G), jnp.int32),
            [pltpu.VMEM((_G, d), jnp.float32)] * _NBUF,
            [pltpu.SemaphoreType.DMA] * _NBUF,
            [pltpu.SemaphoreType.DMA] * _NBUF,
        ],
    )
    def gather(table_hbm, idx_hbm, out_hbm, idx_v, bufs, gsems, ssems):
        wid = lax.axis_index("s") * _NC + lax.axis_index("c")
        pltpu.sync_copy(idx_hbm.at[pl.ds(wid * ng, ng), :], idx_v)

        def out_at(j):
            return out_hbm.at[pl.ds(wid * npw + j * _G, _G), :]

        for b in range(_NBUF):
            pltpu.async_copy(table_hbm.at[idx_v.at[b]], bufs[b], gsems[b])

        scale = jnp.float32(math.sqrt(d))

        def scale_buf(buf):
            def row(i, carry):
                for cc in range(d // 16):
                    sl = (i, pl.ds(cc * 16, 16))
                    buf[sl] = buf[sl] * scale
                return carry

            lax.fori_loop(0, _G, row, 0)

        def outer(k, carry):
            g0 = k * _NBUF
            for b in range(_NBUF):
                j = g0 + b
                pltpu.make_async_copy(
                    table_hbm.at[idx_v.at[j]], bufs[b], gsems[b]).wait()
                scale_buf(bufs[b])
                pltpu.async_copy(bufs[b], out_at(j), ssems[b])
            for b in range(_NBUF):
                j2 = g0 + _NBUF + b

                @pl.when(j2 < ng)
                def _():
                    pltpu.make_async_copy(bufs[b], out_at(g0 + b), ssems[b]).wait()
                    pltpu.async_copy(table_hbm.at[idx_v.at[j2]], bufs[b], gsems[b])

            return carry

        lax.fori_loop(0, ng // _NBUF, outer, 0)

        for b in range(_NBUF):
            pltpu.make_async_copy(bufs[b], out_at(ng - _NBUF + b), ssems[b]).wait()

    return gather


def kernel(tok_ids, emb_table):
    b, l = tok_ids.shape
    v, d = emb_table.shape
    t = b * l
    idx = tok_ids.reshape(t // _G, _G).astype(jnp.int32)
    out = _make_gather(t, d)(emb_table, idx)
    return out.reshape(b, l, d)
